# baseline (speedup 1.0000x reference)
import jax
import jax.numpy as jnp
from jax import lax
from jax.experimental import pallas as pl
from jax.experimental.pallas import tpu as pltpu

_B = 1024
_C = 64
_T = 512
_V = 1000
_BB = 64


def _film_params_kernel(idx_ref, emb_ref, wsc_ref, bsc_ref, wsh_ref, bsh_ref,
                        out_ref):
    idx = idx_ref[0, :]
    iota = lax.broadcasted_iota(jnp.int32, (_B, _V), 1)
    onehot = (idx[:, None] == iota).astype(jnp.float32)
    emb = jnp.dot(onehot, emb_ref[...], preferred_element_type=jnp.float32)
    s1 = lax.dot_general(emb, wsc_ref[...], (((1,), (1,)), ((), ())),
                         preferred_element_type=jnp.float32) + bsc_ref[...]
    sh = lax.dot_general(emb, wsh_ref[...], (((1,), (1,)), ((), ())),
                         preferred_element_type=jnp.float32) + bsh_ref[...]
    out_ref[:, :_C] = s1
    out_ref[:, _C:] = sh


def _film_apply_kernel(p_ref, eeg_ref, out_ref):
    i = pl.program_id(0)
    p = p_ref[i]
    s1 = p[:, :_C]
    sh = p[:, _C:]
    out_ref[...] = eeg_ref[...] * s1[:, :, None] + sh[:, :, None]


def kernel(eeg, subject_idx, emb_table, W_scale, b_scale, W_shift, b_shift):
    idx = subject_idx.astype(jnp.int32).reshape(1, _B)
    bsc = (b_scale + 1.0).reshape(1, _C)
    bsh = b_shift.reshape(1, _C)

    params = pl.pallas_call(
        _film_params_kernel,
        out_shape=jax.ShapeDtypeStruct((_B, 2 * _C), jnp.float32),
    )(idx, emb_table, W_scale, bsc, W_shift, bsh)
    params3 = params.reshape(_B // _BB, _BB, 2 * _C)

    nblk = _B // _BB
    out = pl.pallas_call(
        _film_apply_kernel,
        grid=(nblk,),
        in_specs=[
            pl.BlockSpec((nblk, _BB, 2 * _C), lambda i: (0, 0, 0)),
            pl.BlockSpec((_BB, _C, _T), lambda i: (i, 0, 0)),
        ],
        out_specs=pl.BlockSpec((_BB, _C, _T), lambda i: (i, 0, 0)),
        out_shape=jax.ShapeDtypeStruct((_B, _C, _T), jnp.float32),
        compiler_params=pltpu.CompilerParams(
            dimension_semantics=("arbitrary",)),
    )(params3, eeg)
    return out


# speedup vs baseline: 1.0265x; 1.0265x over previous
import jax
import jax.numpy as jnp
from jax import lax
from jax.experimental import pallas as pl
from jax.experimental.pallas import tpu as pltpu

_B = 1024
_C = 64
_T = 512
_V = 1000
_BB = 64


def _fused_kernel(idx_ref, emb_ref, wsc_ref, bsc_ref, wsh_ref, bsh_ref,
                  eeg_ref, out_ref):
    idx = idx_ref[0, 0, :]
    iota = lax.broadcasted_iota(jnp.int32, (_BB, _V), 1)
    onehot = (idx[:, None] == iota).astype(jnp.float32)
    emb = jnp.dot(onehot, emb_ref[...], preferred_element_type=jnp.float32)
    scale = lax.dot_general(emb, wsc_ref[...], (((1,), (1,)), ((), ())),
                            preferred_element_type=jnp.float32) + bsc_ref[...]
    shift = lax.dot_general(emb, wsh_ref[...], (((1,), (1,)), ((), ())),
                            preferred_element_type=jnp.float32) + bsh_ref[...]
    s1 = 1.0 + scale
    for j in range(_BB):
        out_ref[j] = (eeg_ref[j] * s1[j, :, None] + shift[j, :, None])


def kernel(eeg, subject_idx, emb_table, W_scale, b_scale, W_shift, b_shift):
    idx = subject_idx.astype(jnp.int32).reshape(_B // _BB, 1, _BB)
    bsc = b_scale.reshape(1, _C)
    bsh = b_shift.reshape(1, _C)

    resident = lambda shape: pl.BlockSpec(shape, lambda i: (0,) * len(shape))
    out = pl.pallas_call(
        _fused_kernel,
        grid=(_B // _BB,),
        in_specs=[
            pl.BlockSpec((1, 1, _BB), lambda i: (i, 0, 0)),
            resident((_V, _C)),
            resident((_C, _C)),
            resident((1, _C)),
            resident((_C, _C)),
            resident((1, _C)),
            pl.BlockSpec((_BB, _C, _T), lambda i: (i, 0, 0)),
        ],
        out_specs=pl.BlockSpec((_BB, _C, _T), lambda i: (i, 0, 0)),
        out_shape=jax.ShapeDtypeStruct((_B, _C, _T), jnp.float32),
        compiler_params=pltpu.CompilerParams(
            dimension_semantics=("arbitrary",)),
    )(idx, emb_table, W_scale, bsc, W_shift, bsh, eeg)
    return out
